# baseline (speedup 1.0000x reference)
import functools

import jax
import jax.numpy as jnp
from jax import lax
from jax.experimental import pallas as pl
from jax.experimental.pallas import tpu as pltpu
from jax.experimental.pallas import tpu_sc as plsc

NC = 2
NS = 16
NW = NC * NS
CHUNK = 128
CW = 16
ROW_BLOCK = 10000

_MESH = dict(core_axis_name="c", subcore_axis_name="s")


def _fill_loop(refs_vals, n):
  def row(r, carry):
    for ref, val in refs_vals:
      w = ref.shape[1]
      for c in range(w // 16):
        ref[r, pl.ds(c * 16, 16)] = jnp.full((16,), val, jnp.float32)
    return carry
  lax.fori_loop(0, n, row, 0)


def _counts_sc(dst_idx, acc_rows, T):
  stripe = acc_rows // NS
  GRP = 8

  @functools.partial(
      pl.kernel,
      out_type=jax.ShapeDtypeStruct((NC, acc_rows, CW), jnp.float32),
      mesh=plsc.VectorSubcoreMesh(**_MESH),
      scratch_types=[
          pltpu.VMEM((T, CHUNK), jnp.int32),
          pltpu.VMEM((CHUNK, CW), jnp.float32),
          pltpu.VMEM((CHUNK, CW), jnp.float32),
          pltpu.VMEM_SHARED((acc_rows, CW), jnp.float32),
          pltpu.SemaphoreType.DMA,
          pltpu.SemaphoreType.DMA,
      ],
      compiler_params=pltpu.CompilerParams(use_tc_tiling_on_sc=False),
  )
  def k(dst_hbm, cout_hbm, idx_d, ones_v, czero_v, cacc, semi, sem):
    cid = lax.axis_index("c")
    sid = lax.axis_index("s")
    wid = sid * NC + cid

    cp_d = pltpu.async_copy(dst_hbm.at[wid], idx_d, semi)
    _fill_loop([(ones_v, 1.0), (czero_v, 0.0)], CHUNK)
    for b in range(stripe // CHUNK):
      pltpu.sync_copy(czero_v, cacc.at[pl.ds(sid * stripe + b * CHUNK, CHUNK)])
    cp_d.wait()
    plsc.subcore_barrier()

    def body(g, carry):
      for b in range(GRP):
        pltpu.async_copy(ones_v, cacc.at[idx_d.at[g * GRP + b]], sem, add=True)
      for b in range(GRP):
        pltpu.make_async_copy(ones_v, cacc.at[idx_d.at[0]], sem).wait()
      return carry
    lax.fori_loop(0, T // GRP, body, 0)

    plsc.subcore_barrier()
    pltpu.sync_copy(cacc.at[pl.ds(sid * stripe, stripe)],
                    cout_hbm.at[cid].at[pl.ds(sid * stripe, stripe)])

  return k(dst_idx)


def _scatter_add_sc(table, dst_idx, src_idx, acc_rows, width, T, nbuf,
                    async_scatter):
  stripe = acc_rows // NS

  @functools.partial(
      pl.kernel,
      out_type=jax.ShapeDtypeStruct((NC, acc_rows, width), jnp.float32),
      mesh=plsc.VectorSubcoreMesh(**_MESH),
      scratch_types=[
          pltpu.VMEM((T, CHUNK), jnp.int32),
          pltpu.VMEM((T, CHUNK), jnp.int32),
      ] + [pltpu.VMEM((CHUNK, width), jnp.float32) for _ in range(nbuf)]
      + [pltpu.VMEM((CHUNK, width), jnp.float32)]
      + [pltpu.VMEM_SHARED((acc_rows, width), jnp.float32)]
      + [pltpu.SemaphoreType.DMA for _ in range(2 * nbuf + 1)],
      compiler_params=pltpu.CompilerParams(use_tc_tiling_on_sc=False),
  )
  def k(table2_hbm, dst_hbm, src_hbm, out_hbm, idx_d, idx_s, *bufs):
    rows = bufs[:nbuf]
    zeros_v = bufs[nbuf]
    acc = bufs[nbuf + 1]
    gsems = bufs[nbuf + 2:2 * nbuf + 2]
    ssems = bufs[2 * nbuf + 2:3 * nbuf + 2]
    semi = bufs[3 * nbuf + 2]
    cid = lax.axis_index("c")
    sid = lax.axis_index("s")
    wid = sid * NC + cid
    table_hbm = table2_hbm.at[cid]

    cp_d = pltpu.async_copy(dst_hbm.at[wid], idx_d, semi)
    cp_s = pltpu.async_copy(src_hbm.at[wid], idx_s, semi)

    _fill_loop([(zeros_v, 0.0)], CHUNK)
    for b in range(stripe // CHUNK):
      pltpu.sync_copy(zeros_v, acc.at[pl.ds(sid * stripe + b * CHUNK, CHUNK)])
    cp_d.wait()
    cp_s.wait()
    plsc.subcore_barrier()

    def issue(t, buf, sem):
      pltpu.async_copy(table_hbm.at[idx_s.at[t]], buf, sem)

    def gdrain(buf, sem):
      pltpu.make_async_copy(table_hbm.at[idx_s.at[0]], buf, sem).wait()

    def fire_scatter(t, buf, sem):
      pltpu.async_copy(buf, acc.at[idx_d.at[t]], sem, add=True)

    def sdrain(buf, sem):
      pltpu.make_async_copy(buf, acc.at[idx_d.at[0]], sem).wait()

    for b in range(nbuf):
      issue(b, rows[b], gsems[b])

    if async_scatter:
      def body(g, carry):
        t0 = g * nbuf
        for b in range(nbuf):
          gdrain(rows[b], gsems[b])
          fire_scatter(t0 + b, rows[b], ssems[b])
        for b in range(nbuf):
          sdrain(rows[b], ssems[b])
          issue(t0 + nbuf + b, rows[b], gsems[b])
        return carry
      lax.fori_loop(0, T // nbuf - 1, body, 0)

      t0 = T - nbuf
      for b in range(nbuf):
        gdrain(rows[b], gsems[b])
        fire_scatter(t0 + b, rows[b], ssems[b])
      for b in range(nbuf):
        sdrain(rows[b], ssems[b])
    else:
      def body(g, carry):
        t0 = g * nbuf
        for b in range(nbuf):
          gdrain(rows[b], gsems[b])
          pltpu.sync_copy(rows[b], acc.at[idx_d.at[t0 + b]], add=True)
          issue(t0 + nbuf + b, rows[b], gsems[b])
        return carry
      lax.fori_loop(0, T // nbuf - 1, body, 0)

      t0 = T - nbuf
      for b in range(nbuf):
        gdrain(rows[b], gsems[b])
        pltpu.sync_copy(rows[b], acc.at[idx_d.at[t0 + b]], add=True)

    plsc.subcore_barrier()
    pltpu.sync_copy(acc.at[pl.ds(sid * stripe, stripe)],
                    out_hbm.at[cid].at[pl.ds(sid * stripe, stripe)])

  return k(table, dst_idx, src_idx)


def _mm1_kernel(x_ref, w0t_ref, w1t_ref, b0_ref, b1_ref, vw0_ref, table_ref):
  x = x_ref[...]
  vw0 = jnp.dot(x, w0t_ref[...], preferred_element_type=jnp.float32)
  vw1 = jnp.dot(x, w1t_ref[...], preferred_element_type=jnp.float32)
  vw0_ref[...] = vw0 + b0_ref[...]
  t = vw1 + b1_ref[...]
  table_ref[...] = jnp.broadcast_to(t[None], (2,) + t.shape)


def _mid_kernel(vw0_ref, p_ref, c_ref, w0t_ref, w1t_ref, b0_ref, b1_ref,
                aux_ref, table_ref):
  p = p_ref[...]
  nbr = p[0] + p[1]
  c = c_ref[...]
  cnt = (c[0] + c[1])[:, 0:1]
  dinv = 1.0 / cnt
  h = jnp.maximum((vw0_ref[...] + nbr) * dinv, 0.0)
  hw0 = jnp.dot(h, w0t_ref[...], preferred_element_type=jnp.float32) + b0_ref[...]
  hw1 = jnp.dot(h, w1t_ref[...], preferred_element_type=jnp.float32) + b1_ref[...]
  r = h.shape[0]
  aux_ref[...] = jnp.concatenate(
      [hw0, dinv, jnp.zeros((r, 4), jnp.float32)], axis=1)
  t = jnp.concatenate([hw1, jnp.zeros((r, 13), jnp.float32)], axis=1)
  table_ref[...] = jnp.broadcast_to(t[None], (2,) + t.shape)


def _final_kernel(aux_ref, p_ref, out_ref):
  p = p_ref[...]
  s = p[0] + p[1]
  out_ref[...] = (aux_ref[:, :3] + s[:, :3]) * aux_ref[:, 3:4]


def kernel(features, w0_1, b0_1, w1_1, b1_1, w0_2, b0_2, w1_2, b1_2, edges):
  V = features.shape[0]
  E = edges.shape[0]
  E2 = 2 * E
  per_worker = -(-E2 // (NW * CHUNK))
  per_worker += (-per_worker) % 8
  pad_len = per_worker * NW * CHUNK - E2
  acc_rows = -(-(V + 1) // (NS * CHUNK)) * NS * CHUNK

  e0 = edges[:, 0]
  e1 = edges[:, 1]
  fill = jnp.arange(pad_len, dtype=jnp.int32)
  dst_idx = jnp.concatenate(
      [e0, e1, V + fill % (acc_rows - V)]).reshape(NW, per_worker, CHUNK)
  src_idx = jnp.concatenate(
      [e1, e0, fill % V]).reshape(NW, per_worker, CHUNK)

  grid = V // ROW_BLOCK

  c1 = _counts_sc(dst_idx, acc_rows, per_worker)

  vw0, table1 = pl.pallas_call(
      _mm1_kernel,
      grid=(grid,),
      in_specs=[
          pl.BlockSpec((ROW_BLOCK, 128), lambda i: (i, 0)),
          pl.BlockSpec((128, 64), lambda i: (0, 0)),
          pl.BlockSpec((128, 64), lambda i: (0, 0)),
          pl.BlockSpec((1, 64), lambda i: (0, 0)),
          pl.BlockSpec((1, 64), lambda i: (0, 0)),
      ],
      out_specs=[
          pl.BlockSpec((ROW_BLOCK, 64), lambda i: (i, 0)),
          pl.BlockSpec((NC, ROW_BLOCK, 64), lambda i: (0, i, 0)),
      ],
      out_shape=[
          jax.ShapeDtypeStruct((V, 64), jnp.float32),
          jax.ShapeDtypeStruct((NC, V, 64), jnp.float32),
      ],
  )(features, w0_1.T, w1_1.T, b0_1[None, :], b1_1[None, :])

  p1 = _scatter_add_sc(table1, dst_idx, src_idx, acc_rows, 64, per_worker, 4, False)

  aux, table2 = pl.pallas_call(
      _mid_kernel,
      grid=(grid,),
      in_specs=[
          pl.BlockSpec((ROW_BLOCK, 64), lambda i: (i, 0)),
          pl.BlockSpec((NC, ROW_BLOCK, 64), lambda i: (0, i, 0)),
          pl.BlockSpec((NC, ROW_BLOCK, CW), lambda i: (0, i, 0)),
          pl.BlockSpec((64, 3), lambda i: (0, 0)),
          pl.BlockSpec((64, 3), lambda i: (0, 0)),
          pl.BlockSpec((1, 3), lambda i: (0, 0)),
          pl.BlockSpec((1, 3), lambda i: (0, 0)),
      ],
      out_specs=[
          pl.BlockSpec((ROW_BLOCK, 8), lambda i: (i, 0)),
          pl.BlockSpec((NC, ROW_BLOCK, 16), lambda i: (0, i, 0)),
      ],
      out_shape=[
          jax.ShapeDtypeStruct((V, 8), jnp.float32),
          jax.ShapeDtypeStruct((NC, V, 16), jnp.float32),
      ],
  )(vw0, p1, c1, w0_2.T, w1_2.T, b0_2[None, :], b1_2[None, :])

  p2 = _scatter_add_sc(table2, dst_idx, src_idx, acc_rows, 16, per_worker, 8, True)

  out = pl.pallas_call(
      _final_kernel,
      grid=(grid,),
      in_specs=[
          pl.BlockSpec((ROW_BLOCK, 8), lambda i: (i, 0)),
          pl.BlockSpec((NC, ROW_BLOCK, 16), lambda i: (0, i, 0)),
      ],
      out_specs=pl.BlockSpec((ROW_BLOCK, 3), lambda i: (i, 0)),
      out_shape=jax.ShapeDtypeStruct((V, 3), jnp.float32),
  )(aux, p2)
  return out


# speedup vs baseline: 1.0146x; 1.0146x over previous
import functools

import jax
import jax.numpy as jnp
from jax import lax
from jax.experimental import pallas as pl
from jax.experimental.pallas import tpu as pltpu
from jax.experimental.pallas import tpu_sc as plsc

NC = 2
NS = 16
NW = NC * NS
CHUNK = 128
CW = 16
ROW_BLOCK = 10000

_MESH = dict(core_axis_name="c", subcore_axis_name="s")


def _fill_loop(refs_vals, n):
  def row(r, carry):
    for ref, val in refs_vals:
      w = ref.shape[1]
      for c in range(w // 16):
        ref[r, pl.ds(c * 16, 16)] = jnp.full((16,), val, jnp.float32)
    return carry
  lax.fori_loop(0, n, row, 0)


def _counts_sc(dst_idx, acc_rows, T):
  stripe = acc_rows // NS
  GRP = 8

  @functools.partial(
      pl.kernel,
      out_type=jax.ShapeDtypeStruct((NC, acc_rows, CW), jnp.float32),
      mesh=plsc.VectorSubcoreMesh(**_MESH),
      scratch_types=[
          pltpu.VMEM((T, CHUNK), jnp.int32),
          pltpu.VMEM((CHUNK, CW), jnp.float32),
          pltpu.VMEM((CHUNK, CW), jnp.float32),
          pltpu.VMEM_SHARED((acc_rows, CW), jnp.float32),
          pltpu.SemaphoreType.DMA,
          pltpu.SemaphoreType.DMA,
      ],
      compiler_params=pltpu.CompilerParams(use_tc_tiling_on_sc=False),
  )
  def k(dst_hbm, cout_hbm, idx_d, ones_v, czero_v, cacc, semi, sem):
    cid = lax.axis_index("c")
    sid = lax.axis_index("s")
    wid = sid * NC + cid

    cp_d = pltpu.async_copy(dst_hbm.at[wid], idx_d, semi)
    _fill_loop([(ones_v, 1.0), (czero_v, 0.0)], CHUNK)
    for b in range(stripe // CHUNK):
      pltpu.sync_copy(czero_v, cacc.at[pl.ds(sid * stripe + b * CHUNK, CHUNK)])
    cp_d.wait()
    plsc.subcore_barrier()

    def body(g, carry):
      for b in range(GRP):
        pltpu.async_copy(ones_v, cacc.at[idx_d.at[g * GRP + b]], sem, add=True)
      for b in range(GRP):
        pltpu.make_async_copy(ones_v, cacc.at[idx_d.at[0]], sem).wait()
      return carry
    lax.fori_loop(0, T // GRP, body, 0)

    plsc.subcore_barrier()
    pltpu.sync_copy(cacc.at[pl.ds(sid * stripe, stripe)],
                    cout_hbm.at[cid].at[pl.ds(sid * stripe, stripe)])

  return k(dst_idx)


def _scatter_add_sc(table, dst_idx, src_idx, acc_rows, width, T, nbuf,
                    async_scatter):
  stripe = acc_rows // NS

  @functools.partial(
      pl.kernel,
      out_type=jax.ShapeDtypeStruct((NC, acc_rows, width), jnp.float32),
      mesh=plsc.VectorSubcoreMesh(**_MESH),
      scratch_types=[
          pltpu.VMEM((T, CHUNK), jnp.int32),
          pltpu.VMEM((T, CHUNK), jnp.int32),
      ] + [pltpu.VMEM((CHUNK, width), jnp.float32) for _ in range(nbuf)]
      + [pltpu.VMEM((CHUNK, width), jnp.float32)]
      + [pltpu.VMEM_SHARED((acc_rows, width), jnp.float32)]
      + [pltpu.SemaphoreType.DMA for _ in range(2 * nbuf + 1)],
      compiler_params=pltpu.CompilerParams(use_tc_tiling_on_sc=False),
  )
  def k(table_hbm, dst_hbm, src_hbm, out_hbm, idx_d, idx_s, *bufs):
    rows = bufs[:nbuf]
    zeros_v = bufs[nbuf]
    acc = bufs[nbuf + 1]
    gsems = bufs[nbuf + 2:2 * nbuf + 2]
    ssems = bufs[2 * nbuf + 2:3 * nbuf + 2]
    semi = bufs[3 * nbuf + 2]
    cid = lax.axis_index("c")
    sid = lax.axis_index("s")
    wid = sid * NC + cid

    cp_d = pltpu.async_copy(dst_hbm.at[wid], idx_d, semi)
    cp_s = pltpu.async_copy(src_hbm.at[wid], idx_s, semi)

    _fill_loop([(zeros_v, 0.0)], CHUNK)
    for b in range(stripe // CHUNK):
      pltpu.sync_copy(zeros_v, acc.at[pl.ds(sid * stripe + b * CHUNK, CHUNK)])
    cp_d.wait()
    cp_s.wait()
    plsc.subcore_barrier()

    def issue(t, buf, sem):
      pltpu.async_copy(table_hbm.at[idx_s.at[t]], buf, sem)

    def gdrain(buf, sem):
      pltpu.make_async_copy(table_hbm.at[idx_s.at[0]], buf, sem).wait()

    def fire_scatter(t, buf, sem):
      pltpu.async_copy(buf, acc.at[idx_d.at[t]], sem, add=True)

    def sdrain(buf, sem):
      pltpu.make_async_copy(buf, acc.at[idx_d.at[0]], sem).wait()

    for b in range(nbuf):
      issue(b, rows[b], gsems[b])

    if async_scatter:
      def body(g, carry):
        t0 = g * nbuf
        for b in range(nbuf):
          gdrain(rows[b], gsems[b])
          fire_scatter(t0 + b, rows[b], ssems[b])
        for b in range(nbuf):
          sdrain(rows[b], ssems[b])
          issue(t0 + nbuf + b, rows[b], gsems[b])
        return carry
      lax.fori_loop(0, T // nbuf - 1, body, 0)

      t0 = T - nbuf
      for b in range(nbuf):
        gdrain(rows[b], gsems[b])
        fire_scatter(t0 + b, rows[b], ssems[b])
      for b in range(nbuf):
        sdrain(rows[b], ssems[b])
    else:
      def body(g, carry):
        t0 = g * nbuf
        for b in range(nbuf):
          gdrain(rows[b], gsems[b])
          pltpu.sync_copy(rows[b], acc.at[idx_d.at[t0 + b]], add=True)
          issue(t0 + nbuf + b, rows[b], gsems[b])
        return carry
      lax.fori_loop(0, T // nbuf - 1, body, 0)

      t0 = T - nbuf
      for b in range(nbuf):
        gdrain(rows[b], gsems[b])
        pltpu.sync_copy(rows[b], acc.at[idx_d.at[t0 + b]], add=True)

    plsc.subcore_barrier()
    pltpu.sync_copy(acc.at[pl.ds(sid * stripe, stripe)],
                    out_hbm.at[cid].at[pl.ds(sid * stripe, stripe)])

  return k(table, dst_idx, src_idx)


def _mm1_kernel(x_ref, w0t_ref, w1t_ref, b0_ref, b1_ref, vw0_ref, table_ref):
  x = x_ref[...]
  vw0 = jnp.dot(x, w0t_ref[...], preferred_element_type=jnp.float32)
  vw1 = jnp.dot(x, w1t_ref[...], preferred_element_type=jnp.float32)
  vw0_ref[...] = vw0 + b0_ref[...]
  table_ref[...] = vw1 + b1_ref[...]


def _mid_kernel(vw0_ref, p_ref, c_ref, w0t_ref, w1t_ref, b0_ref, b1_ref,
                aux_ref, table_ref):
  p = p_ref[...]
  nbr = p[0] + p[1]
  c = c_ref[...]
  cnt = (c[0] + c[1])[:, 0:1]
  dinv = 1.0 / cnt
  h = jnp.maximum((vw0_ref[...] + nbr) * dinv, 0.0)
  hw0 = jnp.dot(h, w0t_ref[...], preferred_element_type=jnp.float32) + b0_ref[...]
  hw1 = jnp.dot(h, w1t_ref[...], preferred_element_type=jnp.float32) + b1_ref[...]
  r = h.shape[0]
  aux_ref[...] = jnp.concatenate(
      [hw0, dinv, jnp.zeros((r, 4), jnp.float32)], axis=1)
  table_ref[...] = jnp.concatenate(
      [hw1, jnp.zeros((r, 13), jnp.float32)], axis=1)


def _final_kernel(aux_ref, p_ref, out_ref):
  p = p_ref[...]
  s = p[0] + p[1]
  out_ref[...] = (aux_ref[:, :3] + s[:, :3]) * aux_ref[:, 3:4]


def kernel(features, w0_1, b0_1, w1_1, b1_1, w0_2, b0_2, w1_2, b1_2, edges):
  V = features.shape[0]
  E = edges.shape[0]
  E2 = 2 * E
  per_worker = -(-E2 // (NW * CHUNK))
  per_worker += (-per_worker) % 8
  pad_len = per_worker * NW * CHUNK - E2
  acc_rows = -(-(V + 1) // (NS * CHUNK)) * NS * CHUNK

  e0 = edges[:, 0]
  e1 = edges[:, 1]
  fill = jnp.arange(pad_len, dtype=jnp.int32)
  dst_idx = jnp.concatenate(
      [e0, e1, V + fill % (acc_rows - V)]).reshape(NW, per_worker, CHUNK)
  src_idx = jnp.concatenate(
      [e1, e0, fill % V]).reshape(NW, per_worker, CHUNK)

  grid = V // ROW_BLOCK

  c1 = _counts_sc(dst_idx, acc_rows, per_worker)

  vw0, table1 = pl.pallas_call(
      _mm1_kernel,
      grid=(grid,),
      in_specs=[
          pl.BlockSpec((ROW_BLOCK, 128), lambda i: (i, 0)),
          pl.BlockSpec((128, 64), lambda i: (0, 0)),
          pl.BlockSpec((128, 64), lambda i: (0, 0)),
          pl.BlockSpec((1, 64), lambda i: (0, 0)),
          pl.BlockSpec((1, 64), lambda i: (0, 0)),
      ],
      out_specs=[
          pl.BlockSpec((ROW_BLOCK, 64), lambda i: (i, 0)),
          pl.BlockSpec((ROW_BLOCK, 64), lambda i: (i, 0)),
      ],
      out_shape=[
          jax.ShapeDtypeStruct((V, 64), jnp.float32),
          jax.ShapeDtypeStruct((V, 64), jnp.float32),
      ],
  )(features, w0_1.T, w1_1.T, b0_1[None, :], b1_1[None, :])

  p1 = _scatter_add_sc(table1, dst_idx, src_idx, acc_rows, 64, per_worker, 4, False)

  aux, table2 = pl.pallas_call(
      _mid_kernel,
      grid=(grid,),
      in_specs=[
          pl.BlockSpec((ROW_BLOCK, 64), lambda i: (i, 0)),
          pl.BlockSpec((NC, ROW_BLOCK, 64), lambda i: (0, i, 0)),
          pl.BlockSpec((NC, ROW_BLOCK, CW), lambda i: (0, i, 0)),
          pl.BlockSpec((64, 3), lambda i: (0, 0)),
          pl.BlockSpec((64, 3), lambda i: (0, 0)),
          pl.BlockSpec((1, 3), lambda i: (0, 0)),
          pl.BlockSpec((1, 3), lambda i: (0, 0)),
      ],
      out_specs=[
          pl.BlockSpec((ROW_BLOCK, 8), lambda i: (i, 0)),
          pl.BlockSpec((ROW_BLOCK, 16), lambda i: (i, 0)),
      ],
      out_shape=[
          jax.ShapeDtypeStruct((V, 8), jnp.float32),
          jax.ShapeDtypeStruct((V, 16), jnp.float32),
      ],
  )(vw0, p1, c1, w0_2.T, w1_2.T, b0_2[None, :], b1_2[None, :])

  p2 = _scatter_add_sc(table2, dst_idx, src_idx, acc_rows, 16, per_worker, 8, True)

  out = pl.pallas_call(
      _final_kernel,
      grid=(grid,),
      in_specs=[
          pl.BlockSpec((ROW_BLOCK, 8), lambda i: (i, 0)),
          pl.BlockSpec((NC, ROW_BLOCK, 16), lambda i: (0, i, 0)),
      ],
      out_specs=pl.BlockSpec((ROW_BLOCK, 3), lambda i: (i, 0)),
      out_shape=jax.ShapeDtypeStruct((V, 3), jnp.float32),
  )(aux, p2)
  return out
